# Initial kernel scaffold; baseline (speedup 1.0000x reference)
import jax
import jax.numpy as jnp
from jax.experimental import pallas as pl


def kernel(x, edge_index, edge_attr, batch, atom_tables, bond_tables, eps, W1, b1, W2, b2, Wv1, bv1, Wv2, bv2, vn_emb, Wp, bp):
    raise NotImplementedError("write your pallas kernel here")


# speedup vs baseline: 1.1477x; 1.1477x over previous
import functools

import jax
import jax.numpy as jnp
from jax.experimental import pallas as pl

N = 50000
E = 800000
EMB = 100
L = 5
TASKS = 128
G = 512

BN = 2000


def _mlp_body(hl_ref, agg_ref, eps_ref, w1_ref, b1_ref, w2_ref, b2_ref, out_ref,
              *, final_relu):
    z = (1.0 + eps_ref[0, 0]) * hl_ref[...] + agg_ref[...]
    mid = jnp.maximum(jnp.dot(z, w1_ref[...], preferred_element_type=jnp.float32)
                      + b1_ref[...], 0.0)
    out = jnp.dot(mid, w2_ref[...], preferred_element_type=jnp.float32) + b2_ref[...]
    if final_relu:
        out = jnp.maximum(out, 0.0)
    out_ref[...] = out


def _mlp(hl, agg, eps_l, w1, b1, w2, b2, final_relu):
    n = hl.shape[0]
    d_in = w1.shape[0]
    d_mid = w1.shape[1]
    d_out = w2.shape[1]
    grid = (n // BN,)
    return pl.pallas_call(
        functools.partial(_mlp_body, final_relu=final_relu),
        grid=grid,
        in_specs=[
            pl.BlockSpec((BN, d_in), lambda i: (i, 0)),
            pl.BlockSpec((BN, d_in), lambda i: (i, 0)),
            pl.BlockSpec((1, 1), lambda i: (0, 0)),
            pl.BlockSpec((d_in, d_mid), lambda i: (0, 0)),
            pl.BlockSpec((1, d_mid), lambda i: (0, 0)),
            pl.BlockSpec((d_mid, d_out), lambda i: (0, 0)),
            pl.BlockSpec((1, d_out), lambda i: (0, 0)),
        ],
        out_specs=pl.BlockSpec((BN, d_out), lambda i: (i, 0)),
        out_shape=jax.ShapeDtypeStruct((n, d_out), jnp.float32),
    )(hl, agg, eps_l.reshape(1, 1), w1, b1.reshape(1, -1), w2, b2.reshape(1, -1))


def kernel(x, edge_index, edge_attr, batch, atom_tables, bond_tables, eps,
           W1, b1, W2, b2, Wv1, bv1, Wv2, bv2, vn_emb, Wp, bp):
    h = jnp.zeros((N, EMB), dtype=jnp.float32)
    for i in range(9):
        h = h + jnp.take(atom_tables[i], x[:, i], axis=0)
    vn = jnp.broadcast_to(vn_emb[0], (G, EMB))
    src = edge_index[0]
    dst = edge_index[1]
    for l in range(L):
        hl = h + vn[batch]
        ee = jnp.zeros((E, EMB), dtype=jnp.float32)
        for i in range(3):
            ee = ee + jnp.take(bond_tables[l, i], edge_attr[:, i], axis=0)
        msg = jax.nn.relu(hl[src] + ee)
        agg = jax.ops.segment_sum(msg, dst, num_segments=N)
        out = _mlp(hl, agg, eps[l], W1[l], b1[l], W2[l], b2[l],
                   final_relu=(l < L - 1))
        if l < L - 1:
            h = out
            vt = jax.ops.segment_sum(hl, batch, num_segments=G) + vn
            vn = jax.nn.relu(jax.nn.relu(vt @ Wv1[l] + bv1[l]) @ Wv2[l] + bv2[l])
        else:
            h = out
    counts = jax.ops.segment_sum(jnp.ones((N,), dtype=jnp.float32), batch,
                                 num_segments=G)
    gmean = jax.ops.segment_sum(h, batch, num_segments=G) / jnp.clip(
        counts, 1.0)[:, None]
    return gmean @ Wp + bp


# speedup vs baseline: 3.1170x; 2.7159x over previous
import functools

import jax
import jax.numpy as jnp
from jax import lax
from jax.experimental import pallas as pl
from jax.experimental.pallas import tpu as pltpu
from jax.experimental.pallas import tpu_sc as plsc

N = 50000
E = 800000
EMB = 100
L = 5
TASKS = 128
G = 512

D = 128
DB = 16
NFB = 8
NSC = 2
NTS = 16

BN = 2000
BE = 80000

SB = 128
RPT = 392
EP = NTS * RPT * SB
NSUB = 8
CK = NSUB * SB
NCHUNK = RPT // NSUB
NP = 51200
WBT = NP // NTS
ZB = 640


def _edge_body(hl_ref, src_ref, dst_ref, cidx_ref, comb_ref, out_ref,
               src_v, dst_v, cidx_v, rows_v, comb_v, zero_v, acc_sh, gsem):
    c = lax.axis_index("c")
    s = lax.axis_index("s")

    def zfill(i, _):
        zero_v[i, pl.ds(0, 16)] = jnp.zeros((16,), jnp.float32)
        return 0
    lax.fori_loop(0, ZB, zfill, 0, unroll=4)

    for p in range(NFB // NSC):
        fb = p * 2 + c
        pltpu.sync_copy(comb_ref.at[fb], comb_v)
        for j in range(WBT // ZB):
            pltpu.sync_copy(zero_v, acc_sh.at[pl.ds(s * WBT + j * ZB, ZB)])
        plsc.subcore_barrier()

        def chunk(k, _):
            rbase = s * RPT + k * NSUB
            pltpu.sync_copy(src_ref.at[pl.ds(rbase, NSUB)], src_v)
            pltpu.sync_copy(dst_ref.at[pl.ds(rbase, NSUB)], dst_v)
            pltpu.sync_copy(cidx_ref.at[pl.ds(rbase * SB, CK)], cidx_v)

            def fidx(j, _):
                for i in range(SB // 16):
                    sl = pl.ds(i * 16, 16)
                    src_v[j, sl] = src_v[j, sl] * NFB + fb
                return 0
            lax.fori_loop(0, NSUB, fidx, 0)

            cps = [pltpu.async_copy(hl_ref.at[src_v.at[j]],
                                    rows_v.at[pl.ds(j * SB, SB)], gsem)
                   for j in range(NSUB)]
            for cp in cps:
                cp.wait()

            def eop(e16, _):
                cv = cidx_v[pl.ds(e16 * 16, 16)]
                lo = pl.ds(0, 16)
                for i in range(16):
                    ci = cv[i]
                    e = e16 * 16 + i
                    rows_v[e, lo] = jnp.maximum(
                        rows_v[e, lo] + comb_v[ci, lo], 0.0)
                return 0
            lax.fori_loop(0, CK // 16, eop, 0)

            for j in range(NSUB):
                pltpu.sync_copy(rows_v.at[pl.ds(j * SB, SB)],
                                acc_sh.at[dst_v.at[j]], add=True)
            return 0
        lax.fori_loop(0, NCHUNK, chunk, 0)

        plsc.subcore_barrier()
        pltpu.sync_copy(acc_sh.at[pl.ds(s * WBT, WBT)],
                        out_ref.at[fb, pl.ds(s * WBT, WBT)])
        plsc.subcore_barrier()


_edge_call = pl.kernel(
    _edge_body,
    out_type=jax.ShapeDtypeStruct((NFB, NP, DB), jnp.float32),
    mesh=plsc.VectorSubcoreMesh(core_axis_name="c", subcore_axis_name="s"),
    compiler_params=pltpu.CompilerParams(use_tc_tiling_on_sc=False),
    scratch_types=[
        pltpu.VMEM((NSUB, SB), jnp.int32),
        pltpu.VMEM((NSUB, SB), jnp.int32),
        pltpu.VMEM((CK,), jnp.int32),
        pltpu.VMEM((CK, DB), jnp.float32),
        pltpu.VMEM((128, DB), jnp.float32),
        pltpu.VMEM((ZB, DB), jnp.float32),
        pltpu.VMEM_SHARED((NP, DB), jnp.float32),
        pltpu.SemaphoreType.DMA,
    ],
)


def _mlp_body(hl_ref, agg_ref, eps_ref, w1_ref, b1_ref, w2_ref, b2_ref, out_ref,
              *, final_relu):
    z = (1.0 + eps_ref[0, 0]) * hl_ref[...] + agg_ref[...]
    mid = jnp.maximum(jnp.dot(z, w1_ref[...], preferred_element_type=jnp.float32)
                      + b1_ref[...], 0.0)
    out = jnp.dot(mid, w2_ref[...], preferred_element_type=jnp.float32) + b2_ref[...]
    if final_relu:
        out = jnp.maximum(out, 0.0)
    out_ref[...] = out


def _mlp(hl, agg, eps_l, w1, b1, w2, b2, final_relu):
    n = hl.shape[0]
    d_in = w1.shape[0]
    d_mid = w1.shape[1]
    d_out = w2.shape[1]
    return pl.pallas_call(
        functools.partial(_mlp_body, final_relu=final_relu),
        grid=(n // BN,),
        in_specs=[
            pl.BlockSpec((BN, d_in), lambda i: (i, 0)),
            pl.BlockSpec((BN, d_in), lambda i: (i, 0)),
            pl.BlockSpec((1, 1), lambda i: (0, 0)),
            pl.BlockSpec((d_in, d_mid), lambda i: (0, 0)),
            pl.BlockSpec((1, d_mid), lambda i: (0, 0)),
            pl.BlockSpec((d_mid, d_out), lambda i: (0, 0)),
            pl.BlockSpec((1, d_out), lambda i: (0, 0)),
        ],
        out_specs=pl.BlockSpec((BN, d_out), lambda i: (i, 0)),
        out_shape=jax.ShapeDtypeStruct((n, d_out), jnp.float32),
    )(hl, agg, eps_l.reshape(1, 1), w1, b1.reshape(1, -1), w2, b2.reshape(1, -1))


def _cidx_body(attr_ref, out_ref):
    a = attr_ref[...]
    out_ref[...] = (a[0] + 5 * a[1] + 25 * a[2]).reshape(1, 1, BE)


def _cidx(edge_attr):
    out = pl.pallas_call(
        _cidx_body,
        grid=(E // BE,),
        in_specs=[pl.BlockSpec((3, BE), lambda i: (0, i))],
        out_specs=pl.BlockSpec((1, 1, BE), lambda i: (i, 0, 0)),
        out_shape=jax.ShapeDtypeStruct((E // BE, 1, BE), jnp.int32),
    )(edge_attr.T)
    return out.reshape(E)


def _comb_body(bt_ref, out_ref):
    bt = bt_ref[0]
    cc = lax.broadcasted_iota(jnp.int32, (128, 8), 0)
    jj = lax.broadcasted_iota(jnp.int32, (128, 8), 1)
    oh0 = ((cc % 5) == jj).astype(jnp.float32)
    oh1 = (((cc // 5) % 5) == jj).astype(jnp.float32)
    oh2 = ((cc // 25) == jj).astype(jnp.float32)
    out_ref[0] = (jnp.dot(oh0, bt[0], preferred_element_type=jnp.float32)
                  + jnp.dot(oh1, bt[1], preferred_element_type=jnp.float32)
                  + jnp.dot(oh2, bt[2], preferred_element_type=jnp.float32))


def _comb(bond_pad):
    return pl.pallas_call(
        _comb_body,
        grid=(L,),
        in_specs=[pl.BlockSpec((1, 3, 8, D), lambda i: (i, 0, 0, 0))],
        out_specs=pl.BlockSpec((1, 128, D), lambda i: (i, 0, 0)),
        out_shape=jax.ShapeDtypeStruct((L, 128, D), jnp.float32),
    )(bond_pad)


def kernel(x, edge_index, edge_attr, batch, atom_tables, bond_tables, eps,
           W1, b1, W2, b2, Wv1, bv1, Wv2, bv2, vn_emb, Wp, bp):
    pad = D - EMB
    atom_pad = jnp.pad(atom_tables, ((0, 0), (0, 0), (0, pad)))
    bond_pad = jnp.pad(bond_tables, ((0, 0), (0, 0), (0, 0), (0, pad)))
    W1p = jnp.pad(W1, ((0, 0), (0, pad), (0, 0)))
    W2p = jnp.pad(W2, ((0, 0), (0, 0), (0, pad)))
    b2p = jnp.pad(b2, ((0, 0), (0, pad)))
    Wv1p = jnp.pad(Wv1, ((0, 0), (0, pad), (0, 0)))
    Wv2p = jnp.pad(Wv2, ((0, 0), (0, 0), (0, pad)))
    bv2p = jnp.pad(bv2, ((0, 0), (0, pad)))
    Wpp = jnp.pad(Wp, ((0, pad), (0, 0)))
    vn0 = jnp.pad(vn_emb, ((0, 0), (0, pad)))

    h = jnp.zeros((N, D), dtype=jnp.float32)
    for i in range(9):
        h = h + jnp.take(atom_pad[i], x[:, i], axis=0)

    vn = jnp.broadcast_to(vn0[0], (G, D))
    npad = EP - E
    src2 = jnp.concatenate(
        [edge_index[0], jnp.zeros((npad,), jnp.int32)]).reshape(EP // SB, SB)
    dst2 = jnp.concatenate(
        [edge_index[1], jnp.full((npad,), N, jnp.int32)]).reshape(EP // SB, SB)
    cidxf = jnp.concatenate([_cidx(edge_attr), jnp.zeros((npad,), jnp.int32)])
    comb = _comb(bond_pad)
    comb4 = comb.reshape(L, 128, NFB, DB).transpose(0, 2, 1, 3)

    for l in range(L):
        hl = h + vn[batch]
        agg4 = _edge_call(hl.reshape(N * NFB, DB), src2, dst2, cidxf, comb4[l])
        agg = agg4[:, :N].transpose(1, 0, 2).reshape(N, D)
        out = _mlp(hl, agg, eps[l], W1p[l], b1[l], W2p[l], b2p[l],
                   final_relu=(l < L - 1))
        if l < L - 1:
            h = out
            vt = jax.ops.segment_sum(hl, batch, num_segments=G) + vn
            vn = jax.nn.relu(jax.nn.relu(vt @ Wv1p[l] + bv1[l]) @ Wv2p[l]
                             + bv2p[l])
        else:
            h = out
    counts = jax.ops.segment_sum(jnp.ones((N,), dtype=jnp.float32), batch,
                                 num_segments=G)
    gmean = jax.ops.segment_sum(h, batch, num_segments=G) / jnp.clip(
        counts, 1.0)[:, None]
    return gmean @ Wpp + bp


# speedup vs baseline: 3.5048x; 1.1244x over previous
import functools

import jax
import jax.numpy as jnp
from jax import lax
from jax.experimental import pallas as pl
from jax.experimental.pallas import tpu as pltpu
from jax.experimental.pallas import tpu_sc as plsc

N = 50000
E = 800000
EMB = 100
L = 5
TASKS = 128
G = 512

D = 128
DB = 16
NFB = 8
NSC = 2
NTS = 16

BN = 2000
BE = 80000

SB = 128
RPT = 400
EP = NTS * RPT * SB
NSUB = 8
CK = NSUB * SB
CPW = CK // 4
NCHUNK = RPT // NSUB
NP = 51200
WBT = NP // NTS
ZB = 640


def _edge_body(hl_ref, meta_ref, cidxp_ref, comb_ref, out_ref,
               meta_a, meta_b, cp_a, cp_b, rows_a, rows_b,
               comb_v, zero_v, acc_sh, gsem, msem):
    c = lax.axis_index("c")
    s = lax.axis_index("s")

    def zfill(i, _):
        zero_v[i, pl.ds(0, 16)] = jnp.zeros((16,), jnp.float32)
        return 0
    lax.fori_loop(0, ZB, zfill, 0, unroll=4)

    def stage_issue(k, mbuf, cbuf):
        rbase = s * RPT + k * NSUB
        pltpu.async_copy(meta_ref.at[pl.ds(rbase, NSUB)], mbuf, msem)
        pltpu.async_copy(cidxp_ref.at[pl.ds((s * RPT // NSUB + k) * CPW, CPW)],
                         cbuf, msem)

    def meta_wait(mbuf, cbuf):
        pltpu.make_async_copy(meta_ref.at[pl.ds(0, NSUB)], mbuf, msem).wait()
        pltpu.make_async_copy(cidxp_ref.at[pl.ds(0, CPW)], cbuf, msem).wait()

    def gather_drain(rbuf):
        pltpu.make_async_copy(hl_ref.at[pl.ds(0, CK)], rbuf, gsem).wait()

    for p in range(NFB // NSC):
        fb = p * 2 + c
        pltpu.sync_copy(comb_ref.at[fb], comb_v)
        for j in range(WBT // ZB):
            pltpu.sync_copy(zero_v, acc_sh.at[pl.ds(s * WBT + j * ZB, ZB)])
        plsc.subcore_barrier()

        def fidx_fire(mbuf, rbuf):
            def fidx(j, _):
                for i in range(SB // 16):
                    sl = pl.ds(i * 16, 16)
                    mbuf[j, 0, sl] = mbuf[j, 0, sl] * NFB + fb
                return 0
            lax.fori_loop(0, NSUB, fidx, 0)
            for j in range(NSUB):
                pltpu.async_copy(hl_ref.at[mbuf.at[j, 0]],
                                 rbuf.at[pl.ds(j * SB, SB)], gsem)

        def compute_scatter(mbuf, cbuf, rbuf):
            lo = pl.ds(0, 16)

            def eop(g, _):
                pw = cbuf[pl.ds((g % 16) * 16, 16)]
                civ = (pw >> (8 * (g // 16))) & 127
                for i in range(16):
                    ci = civ[i]
                    e = g * 16 + i
                    rbuf[e, lo] = jnp.maximum(
                        rbuf[e, lo] + comb_v[ci, lo], 0.0)
                return 0
            lax.fori_loop(0, CK // 16, eop, 0)
            for j in range(NSUB):
                pltpu.sync_copy(rbuf.at[pl.ds(j * SB, SB)],
                                acc_sh.at[mbuf.at[j, 1]], add=True)

        stage_issue(0, meta_a, cp_a)
        meta_wait(meta_a, cp_a)
        fidx_fire(meta_a, rows_a)

        def pipe(k2, _):
            ka = 2 * k2
            stage_issue(ka + 1, meta_b, cp_b)
            gather_drain(rows_a)
            meta_wait(meta_b, cp_b)
            fidx_fire(meta_b, rows_b)
            compute_scatter(meta_a, cp_a, rows_a)
            stage_issue(ka + 2, meta_a, cp_a)
            gather_drain(rows_b)
            meta_wait(meta_a, cp_a)
            fidx_fire(meta_a, rows_a)
            compute_scatter(meta_b, cp_b, rows_b)
            return 0
        lax.fori_loop(0, NCHUNK // 2 - 1, pipe, 0)

        stage_issue(NCHUNK - 1, meta_b, cp_b)
        gather_drain(rows_a)
        meta_wait(meta_b, cp_b)
        fidx_fire(meta_b, rows_b)
        compute_scatter(meta_a, cp_a, rows_a)
        gather_drain(rows_b)
        compute_scatter(meta_b, cp_b, rows_b)

        plsc.subcore_barrier()
        pltpu.sync_copy(acc_sh.at[pl.ds(s * WBT, WBT)],
                        out_ref.at[fb, pl.ds(s * WBT, WBT)])
        plsc.subcore_barrier()


_edge_call = pl.kernel(
    _edge_body,
    out_type=jax.ShapeDtypeStruct((NFB, NP, DB), jnp.float32),
    mesh=plsc.VectorSubcoreMesh(core_axis_name="c", subcore_axis_name="s"),
    compiler_params=pltpu.CompilerParams(use_tc_tiling_on_sc=False),
    scratch_types=[
        pltpu.VMEM((NSUB, 2, SB), jnp.int32),
        pltpu.VMEM((NSUB, 2, SB), jnp.int32),
        pltpu.VMEM((CPW,), jnp.int32),
        pltpu.VMEM((CPW,), jnp.int32),
        pltpu.VMEM((CK, DB), jnp.float32),
        pltpu.VMEM((CK, DB), jnp.float32),
        pltpu.VMEM((128, DB), jnp.float32),
        pltpu.VMEM((ZB, DB), jnp.float32),
        pltpu.VMEM_SHARED((NP, DB), jnp.float32),
        pltpu.SemaphoreType.DMA,
        pltpu.SemaphoreType.DMA,
    ],
)


def _mlp_body(hl_ref, agg_ref, eps_ref, w1_ref, b1_ref, w2_ref, b2_ref, out_ref,
              *, final_relu):
    z = (1.0 + eps_ref[0, 0]) * hl_ref[...] + agg_ref[...]
    mid = jnp.maximum(jnp.dot(z, w1_ref[...], preferred_element_type=jnp.float32)
                      + b1_ref[...], 0.0)
    out = jnp.dot(mid, w2_ref[...], preferred_element_type=jnp.float32) + b2_ref[...]
    if final_relu:
        out = jnp.maximum(out, 0.0)
    out_ref[...] = out


def _mlp(hl, agg, eps_l, w1, b1, w2, b2, final_relu):
    n = hl.shape[0]
    d_in = w1.shape[0]
    d_mid = w1.shape[1]
    d_out = w2.shape[1]
    return pl.pallas_call(
        functools.partial(_mlp_body, final_relu=final_relu),
        grid=(n // BN,),
        in_specs=[
            pl.BlockSpec((BN, d_in), lambda i: (i, 0)),
            pl.BlockSpec((BN, d_in), lambda i: (i, 0)),
            pl.BlockSpec((1, 1), lambda i: (0, 0)),
            pl.BlockSpec((d_in, d_mid), lambda i: (0, 0)),
            pl.BlockSpec((1, d_mid), lambda i: (0, 0)),
            pl.BlockSpec((d_mid, d_out), lambda i: (0, 0)),
            pl.BlockSpec((1, d_out), lambda i: (0, 0)),
        ],
        out_specs=pl.BlockSpec((BN, d_out), lambda i: (i, 0)),
        out_shape=jax.ShapeDtypeStruct((n, d_out), jnp.float32),
    )(hl, agg, eps_l.reshape(1, 1), w1, b1.reshape(1, -1), w2, b2.reshape(1, -1))


def _cidx_body(attr_ref, out_ref):
    a = attr_ref[...]
    out_ref[...] = (a[0] + 5 * a[1] + 25 * a[2]).reshape(1, 1, BE)


def _cidx(edge_attr):
    out = pl.pallas_call(
        _cidx_body,
        grid=(E // BE,),
        in_specs=[pl.BlockSpec((3, BE), lambda i: (0, i))],
        out_specs=pl.BlockSpec((1, 1, BE), lambda i: (i, 0, 0)),
        out_shape=jax.ShapeDtypeStruct((E // BE, 1, BE), jnp.int32),
    )(edge_attr.T)
    return out.reshape(E)


def _pack_body(c_ref, out_ref):
    cw = c_ref[...]
    out_ref[...] = (cw[:, 0] | (cw[:, 1] << 8) | (cw[:, 2] << 16)
                    | (cw[:, 3] << 24))


def _pack(cidx_pad):
    out = pl.pallas_call(
        _pack_body,
        grid=(EP // 1024 // 40,),
        in_specs=[pl.BlockSpec((40, 4, 256), lambda i: (i, 0, 0))],
        out_specs=pl.BlockSpec((40, 256), lambda i: (i, 0)),
        out_shape=jax.ShapeDtypeStruct((EP // 1024, 256), jnp.int32),
    )(cidx_pad.reshape(EP // 1024, 4, 256))
    return out.reshape(EP // 4)


def _comb_body(bt_ref, out_ref):
    bt = bt_ref[0]
    cc = lax.broadcasted_iota(jnp.int32, (128, 8), 0)
    jj = lax.broadcasted_iota(jnp.int32, (128, 8), 1)
    oh0 = ((cc % 5) == jj).astype(jnp.float32)
    oh1 = (((cc // 5) % 5) == jj).astype(jnp.float32)
    oh2 = ((cc // 25) == jj).astype(jnp.float32)
    out_ref[0] = (jnp.dot(oh0, bt[0], preferred_element_type=jnp.float32)
                  + jnp.dot(oh1, bt[1], preferred_element_type=jnp.float32)
                  + jnp.dot(oh2, bt[2], preferred_element_type=jnp.float32))


def _comb(bond_pad):
    return pl.pallas_call(
        _comb_body,
        grid=(L,),
        in_specs=[pl.BlockSpec((1, 3, 8, D), lambda i: (i, 0, 0, 0))],
        out_specs=pl.BlockSpec((1, 128, D), lambda i: (i, 0, 0)),
        out_shape=jax.ShapeDtypeStruct((L, 128, D), jnp.float32),
    )(bond_pad)


def kernel(x, edge_index, edge_attr, batch, atom_tables, bond_tables, eps,
           W1, b1, W2, b2, Wv1, bv1, Wv2, bv2, vn_emb, Wp, bp):
    pad = D - EMB
    atom_pad = jnp.pad(atom_tables, ((0, 0), (0, 0), (0, pad)))
    bond_pad = jnp.pad(bond_tables, ((0, 0), (0, 0), (0, 0), (0, pad)))
    W1p = jnp.pad(W1, ((0, 0), (0, pad), (0, 0)))
    W2p = jnp.pad(W2, ((0, 0), (0, 0), (0, pad)))
    b2p = jnp.pad(b2, ((0, 0), (0, pad)))
    Wv1p = jnp.pad(Wv1, ((0, 0), (0, pad), (0, 0)))
    Wv2p = jnp.pad(Wv2, ((0, 0), (0, 0), (0, pad)))
    bv2p = jnp.pad(bv2, ((0, 0), (0, pad)))
    Wpp = jnp.pad(Wp, ((0, pad), (0, 0)))
    vn0 = jnp.pad(vn_emb, ((0, 0), (0, pad)))

    h = jnp.zeros((N, D), dtype=jnp.float32)
    for i in range(9):
        h = h + jnp.take(atom_pad[i], x[:, i], axis=0)

    vn = jnp.broadcast_to(vn0[0], (G, D))
    npad = EP - E
    srcp = jnp.concatenate(
        [edge_index[0], jnp.zeros((npad,), jnp.int32)]).reshape(EP // SB, 1, SB)
    dstp = jnp.concatenate(
        [edge_index[1], jnp.full((npad,), N, jnp.int32)]).reshape(EP // SB, 1, SB)
    meta = jnp.concatenate([srcp, dstp], axis=1)
    cidxp = _pack(jnp.concatenate([_cidx(edge_attr),
                                   jnp.zeros((npad,), jnp.int32)]))
    comb = _comb(bond_pad)
    comb4 = comb.reshape(L, 128, NFB, DB).transpose(0, 2, 1, 3)

    for l in range(L):
        hl = h + vn[batch]
        agg4 = _edge_call(hl.reshape(N * NFB, DB), meta, cidxp, comb4[l])
        agg = agg4[:, :N].transpose(1, 0, 2).reshape(N, D)
        out = _mlp(hl, agg, eps[l], W1p[l], b1[l], W2p[l], b2p[l],
                   final_relu=(l < L - 1))
        if l < L - 1:
            h = out
            vt = jax.ops.segment_sum(hl, batch, num_segments=G) + vn
            vn = jax.nn.relu(jax.nn.relu(vt @ Wv1p[l] + bv1[l]) @ Wv2p[l]
                             + bv2p[l])
        else:
            h = out
    counts = jax.ops.segment_sum(jnp.ones((N,), dtype=jnp.float32), batch,
                                 num_segments=G)
    gmean = jax.ops.segment_sum(h, batch, num_segments=G) / jnp.clip(
        counts, 1.0)[:, None]
    return gmean @ Wpp + bp
